# baseline (speedup 1.0000x reference)
import jax
import jax.numpy as jnp
from jax import lax
from jax.experimental import pallas as pl
from jax.experimental.pallas import tpu as pltpu
from jax.experimental.pallas import tpu_sc as plsc

_ALPHA = 0.25
_EPS = 1e-9

_B, _C, _H, _W = 16, 1, 512, 512
_ROWS = _B * _C * _H
_N = float(_ROWS * _W)

_K = 2
_GRID = 8
_SROWS = _ROWS // _GRID


def _tc_body(cls_ref, tgt_ref, reg_ref, out_ref, fa, sa, ta, ca):
    i = pl.program_id(0)

    @pl.when(i == 0)
    def _():
        z = jnp.zeros((8, _W), jnp.float32)
        fa[...] = z
        sa[...] = z
        ta[...] = z
        ca[...] = z

    for j in range(_SROWS // 8):
        rows = pl.ds(j * 8, 8)
        c = cls_ref[rows, :]
        t = tgt_ref[rows, :]
        pos = t > 0.0
        one_m = 1.0 - c
        l1 = jnp.log(c + _EPS)
        l2 = jnp.log(one_m + _EPS)
        fa[...] += jnp.where(pos, (-_ALPHA) * (one_m * one_m) * l1,
                             (_ALPHA - 1.0) * (c * c) * l2)

    @pl.when(i >= _K)
    def _():
        for j in range(_SROWS // 8):
            rows = pl.ds(j * 8, 8)
            t = tgt_ref[rows, :]
            r = reg_ref[rows, :]
            pos = t > 0.0
            d = r - t
            sq = d * d
            sa[...] += jnp.where(pos, sq, 0.0)
            ta[...] += sq
            ca[...] += jnp.where(pos, 1.0, 0.0)

    @pl.when(i == _GRID - 1)
    def _():
        out_ref[0] = jnp.sum(fa[...])
        out_ref[1] = jnp.sum(sa[...])
        out_ref[2] = jnp.sum(ta[...])
        out_ref[3] = jnp.sum(ca[...])


def _tc_part(cls2, tgt2, reg2):
    blk = (_SROWS, _W)
    spec = pl.BlockSpec(blk, lambda i: (i, 0))
    reg_spec = pl.BlockSpec(blk, lambda i: (jnp.maximum(i, _K), 0))
    return pl.pallas_call(
        _tc_body,
        grid=(_GRID,),
        in_specs=[spec, spec, reg_spec],
        out_specs=pl.BlockSpec(memory_space=pltpu.SMEM),
        out_shape=jax.ShapeDtypeStruct((4,), jnp.float32),
        scratch_shapes=[
            pltpu.VMEM((8, _W), jnp.float32),
            pltpu.VMEM((8, _W), jnp.float32),
            pltpu.VMEM((8, _W), jnp.float32),
            pltpu.VMEM((8, _W), jnp.float32),
        ],
        compiler_params=pltpu.CompilerParams(
            dimension_semantics=("arbitrary",)),
    )(cls2, tgt2, reg2)


_NC, _NS, _L = 1, 16, 16
_NW = _NC * _NS
_SC_ROWS = _K * _ROWS // 8
_WROWS = _SC_ROWS // _NW
_CROWS = 32
_NCH = _WROWS // _CROWS
_NBUF = min(3, _NCH)
_PRIME = min(_NBUF - 1, _NCH) if _NBUF > 1 else 1


def _sc_body(reg_ref, tgt_ref, out_ref, rbuf, tbuf, av,
             s0, s1, s2, s3, s4, s5):
    wid = lax.axis_index("s") * _NC + lax.axis_index("c")
    row0 = wid * _WROWS
    rsem = (s0, s1, s2)
    tsem = (s3, s4, s5)

    def start(k, slot):
        rows = pl.ds(row0 + k * _CROWS, _CROWS)
        c1 = pltpu.async_copy(reg_ref.at[rows, :], rbuf.at[slot], rsem[slot])
        c2 = pltpu.async_copy(tgt_ref.at[rows, :], tbuf.at[slot], tsem[slot])
        return c1, c2

    pend = {k: start(k, k % _NBUF) for k in range(_PRIME)}
    zero = jnp.zeros((_L,), jnp.float32)
    accs = tuple(zero for _ in range(12))
    for k in range(_NCH):
        slot = k % _NBUF
        nxt = k + _PRIME
        if nxt < _NCH:
            pend[nxt] = start(nxt, nxt % _NBUF)
        c1, c2 = pend.pop(k)
        c1.wait()
        c2.wait()

        def row_body(rr, a, slot=slot):
            a = list(a)
            for cc in range(_W // _L):
                g = cc % 4
                cols = pl.ds(cc * _L, _L)
                r = rbuf[slot, rr, cols]
                t = tbuf[slot, rr, cols]
                pos = t > 0.0
                d = r - t
                sq = d * d
                a[g] = a[g] + jnp.where(pos, sq, 0.0)
                a[4 + g] = a[4 + g] + sq
                a[8 + g] = a[8 + g] + jnp.where(pos, 1.0, 0.0)
            return tuple(a)

        accs = lax.fori_loop(0, _CROWS, row_body, accs)

    av[0, :] = (accs[0] + accs[1]) + (accs[2] + accs[3])
    av[1, :] = (accs[4] + accs[5]) + (accs[6] + accs[7])
    av[2, :] = (accs[8] + accs[9]) + (accs[10] + accs[11])
    pltpu.sync_copy(av, out_ref.at[wid])


def _sc_part(reg2, tgt2):
    mesh = plsc.VectorSubcoreMesh(
        core_axis_name="c", subcore_axis_name="s",
        num_cores=_NC, num_subcores=_NS)
    fn = pl.kernel(
        _sc_body,
        out_type=jax.ShapeDtypeStruct((_NW, 3, _L), jnp.float32),
        mesh=mesh,
        scratch_types=[
            pltpu.VMEM((_NBUF, _CROWS, _W), jnp.float32),
            pltpu.VMEM((_NBUF, _CROWS, _W), jnp.float32),
            pltpu.VMEM((3, _L), jnp.float32),
            pltpu.SemaphoreType.DMA,
            pltpu.SemaphoreType.DMA,
            pltpu.SemaphoreType.DMA,
            pltpu.SemaphoreType.DMA,
            pltpu.SemaphoreType.DMA,
            pltpu.SemaphoreType.DMA,
        ],
    )
    return fn(reg2, tgt2)


def kernel(cls, reg, targets, epoch):
    cls2 = cls.reshape(_ROWS, _W)
    reg2 = reg.reshape(_ROWS, _W)
    tgt2 = targets.reshape(_ROWS, _W)

    sc_part = _sc_part(reg2, tgt2)
    tc_part = _tc_part(cls2, tgt2, reg2)

    fsq = tc_part[1] + jnp.sum(sc_part[:, 0, :])
    tsq = tc_part[2] + jnp.sum(sc_part[:, 1, :])
    cnt = tc_part[3] + jnp.sum(sc_part[:, 2, :])

    coeff = jnp.where(jnp.asarray(epoch) < 500, 10.0, 0.1).astype(jnp.float32)
    fc = jnp.maximum(cnt, 1.0)
    uc = jnp.maximum(_N - cnt, 1.0)
    loss_cls = tc_part[0] / _N
    lrf = fsq / fc
    lru = (tsq - fsq) / uc
    lr = 20.0 * lrf + lru
    loss = lr + coeff * loss_cls
    return (loss, lr, lrf, lru, loss_cls)


# speedup vs baseline: 2.2911x; 2.2911x over previous
import jax
import jax.numpy as jnp
from jax.experimental import pallas as pl
from jax.experimental.pallas import tpu as pltpu

_ALPHA = 0.25
_EPS = 1e-9

_B, _C, _H, _W = 16, 1, 512, 512
_BB = 4
_GRID = _B // _BB
_N = float(_B * _C * _H * _W)


def _body(coeff_ref, cls_ref, reg_ref, tgt_ref, out_ref, fa, sa, ta, ca):
    i = pl.program_id(0)

    @pl.when(i == 0)
    def _():
        z = jnp.zeros((8, _W), jnp.float32)
        fa[...] = z
        sa[...] = z
        ta[...] = z
        ca[...] = z

    for j in range(_BB * _H // 8):
        rows = pl.ds(j * 8, 8)
        c = cls_ref[rows, :]
        r = reg_ref[rows, :]
        t = tgt_ref[rows, :]
        pos = t > 0.0
        one_m = 1.0 - c
        l1 = jnp.log(c + _EPS)
        l2 = jnp.log(one_m + _EPS)
        focal = jnp.where(pos, (-_ALPHA) * (one_m * one_m) * l1,
                          (_ALPHA - 1.0) * (c * c) * l2)
        d = r - t
        sq = d * d
        fa[...] += focal
        sa[...] += jnp.where(pos, sq, 0.0)
        ta[...] += sq
        ca[...] += jnp.where(pos, 1.0, 0.0)

    @pl.when(i == _GRID - 1)
    def _():
        foc = jnp.sum(fa[...])
        fsq = jnp.sum(sa[...])
        tsq = jnp.sum(ta[...])
        cnt = jnp.sum(ca[...])
        fc = jnp.maximum(cnt, 1.0)
        uc = jnp.maximum(_N - cnt, 1.0)
        loss_cls = foc / _N
        lrf = fsq / fc
        lru = (tsq - fsq) / uc
        lr = 20.0 * lrf + lru
        out_ref[0] = lr + coeff_ref[0] * loss_cls
        out_ref[1] = lr
        out_ref[2] = lrf
        out_ref[3] = lru
        out_ref[4] = loss_cls


def kernel(cls, reg, targets, epoch):
    coeff = jnp.where(jnp.asarray(epoch) < 500, 10.0, 0.1).astype(
        jnp.float32).reshape(1)

    blk = (_BB * _H, _W)
    spec = pl.BlockSpec(blk, lambda i: (i, 0))
    out = pl.pallas_call(
        _body,
        grid=(_GRID,),
        in_specs=[pl.BlockSpec(memory_space=pltpu.SMEM), spec, spec, spec],
        out_specs=pl.BlockSpec(memory_space=pltpu.SMEM),
        out_shape=jax.ShapeDtypeStruct((5,), jnp.float32),
        scratch_shapes=[
            pltpu.VMEM((8, _W), jnp.float32),
            pltpu.VMEM((8, _W), jnp.float32),
            pltpu.VMEM((8, _W), jnp.float32),
            pltpu.VMEM((8, _W), jnp.float32),
        ],
        compiler_params=pltpu.CompilerParams(
            dimension_semantics=("arbitrary",)),
    )(coeff, cls.reshape(_B * _H, _W), reg.reshape(_B * _H, _W),
      targets.reshape(_B * _H, _W))

    return (out[0], out[1], out[2], out[3], out[4])
